# baseline (speedup 1.0000x reference)
import functools

import jax
import jax.numpy as jnp
from jax import lax
from jax.experimental import pallas as pl
from jax.experimental.pallas import tpu as pltpu
from jax.experimental.pallas import tpu_sc as plsc

N_NODES = 10000
N_EDGES = 320000
NODE_DIM = 128
EDGE_DIM = 16
HIDDEN_DIM = 32

NC = 2
NS = 16
NW = NC * NS
CH = 128
NT = N_EDGES // CH
MAXCH = 79
IDX_PRE = MAXCH * CH
NBUF = 8
AHEAD = 4
NOUTER = 10
QROWS = N_EDGES // 4
QTILES = NT // 4


def _proj_body(nf_ref, wa_ref, wb_ref, pa_ref, pb_ref):
    nf = nf_ref[...]
    pa_ref[...] = jnp.dot(nf, wa_ref[...], preferred_element_type=jnp.float32)
    pb_ref[...] = jnp.dot(nf, wb_ref[...], preferred_element_type=jnp.float32)


_proj_call = pl.pallas_call(
    _proj_body,
    out_shape=[
        jax.ShapeDtypeStruct((N_NODES, HIDDEN_DIM), jnp.float32),
        jax.ShapeDtypeStruct((N_NODES, HIDDEN_DIM), jnp.float32),
    ],
)


_mesh = plsc.VectorSubcoreMesh(
    core_axis_name="c", subcore_axis_name="s", num_cores=NC, num_subcores=NS
)


@functools.partial(
    pl.kernel,
    out_type=[
        jax.ShapeDtypeStruct((QROWS, 4 * HIDDEN_DIM), jnp.float32),
        jax.ShapeDtypeStruct((QROWS, 4 * HIDDEN_DIM), jnp.float32),
    ],
    mesh=_mesh,
    scratch_types=[
        pltpu.VMEM((IDX_PRE,), jnp.int32),
        pltpu.VMEM((IDX_PRE,), jnp.int32),
        pltpu.VMEM((NBUF, CH, HIDDEN_DIM), jnp.float32),
        pltpu.VMEM((NBUF, CH, HIDDEN_DIM), jnp.float32),
        [pltpu.SemaphoreType.DMA] * NBUF,
        [pltpu.SemaphoreType.DMA] * NBUF,
        [pltpu.SemaphoreType.DMA] * NBUF,
    ],
    compiler_params=pltpu.CompilerParams(use_tc_tiling_on_sc=False,
                                         needs_layout_passes=False),
)
def _gather2(pa_hbm, pb_hbm, src_hbm, dst_hbm, ga_hbm, gb_hbm,
             src_all, dst_all, a_v, b_v, gsems, asems, bsems):
    wid = lax.axis_index("s") * NC + lax.axis_index("c")
    t0 = wid * NT // NW
    t1 = (wid + 1) * NT // NW
    nch = t1 - t0
    q = wid // 8
    r0base = (t0 - q * QTILES) * CH

    pltpu.sync_copy(src_hbm.at[pl.ds(t0 * CH, IDX_PRE)], src_all)
    pltpu.sync_copy(dst_hbm.at[pl.ds(t0 * CH, IDX_PRE)], dst_all)

    def gathers(i, b):
        loc = i * CH
        ca = pltpu.make_async_copy(
            pa_hbm.at[src_all.at[pl.ds(loc, CH)]], a_v.at[b], gsems[b])
        cb = pltpu.make_async_copy(
            pb_hbm.at[dst_all.at[pl.ds(loc, CH)]], b_v.at[b], gsems[b])
        return ca, cb

    def stores(i, b):
        r0 = r0base + i * CH
        col = pl.ds(q * HIDDEN_DIM, HIDDEN_DIM)
        sa = pltpu.make_async_copy(
            a_v.at[b], ga_hbm.at[pl.ds(r0, CH), col], asems[b])
        sb = pltpu.make_async_copy(
            b_v.at[b], gb_hbm.at[pl.ds(r0, CH), col], bsems[b])
        return sa, sb

    for b in range(AHEAD):
        ca, cb = gathers(b, b)
        ca.start()
        cb.start()

    def outer(t, carry):
        for b in range(NBUF):
            i = t * NBUF + b

            @pl.when(i < nch)
            def _():
                ca, cb = gathers(i, b)
                ca.wait()
                cb.wait()
                sa, sb = stores(i, b)
                sa.start()
                sb.start()

            @pl.when(i + AHEAD < nch)
            def _():
                @pl.when(i >= AHEAD)
                def _():
                    sa, sb = stores(i - AHEAD, (b + AHEAD) % NBUF)
                    sa.wait()
                    sb.wait()

                na, nb = gathers(i + AHEAD, (b + AHEAD) % NBUF)
                na.start()
                nb.start()

        return carry

    lax.fori_loop(0, NOUTER, outer, 0)

    for b in range(NBUF):
        sa, sb = stores(0, b)
        sa.wait()
        sb.wait()


_GR = 3200
_NBLK = QROWS // _GR


def _mlp_body(ga_ref, gb_ref, e0_ref, e1_ref, e2_ref, e3_ref, eye_ref,
              w1et_ref, b1_ref, w2t_ref, b2_ref,
              o0_ref, o1_ref, o2_ref, o3_ref):
    gs = ga_ref[...] + gb_ref[...]
    gt = gs.T
    w1et = w1et_ref[...]
    w2t = w2t_ref[...]
    b1 = b1_ref[...]
    b2 = b2_ref[...]
    for c, (e_ref, o_ref) in enumerate(
            [(e0_ref, o0_ref), (e1_ref, o1_ref), (e2_ref, o2_ref),
             (e3_ref, o3_ref)]):
        ec = e_ref[...]
        pre = gt[c * HIDDEN_DIM:(c + 1) * HIDDEN_DIM, :] + jnp.dot(
            w1et, ec, preferred_element_type=jnp.float32) + b1
        h = jnp.maximum(pre, 0.0)
        o_ref[...] = ec + jnp.dot(w2t, h,
                                  preferred_element_type=jnp.float32) + b2


def _espec(c):
    return pl.BlockSpec((EDGE_DIM, _GR), lambda i, c=c: (0, i + c * _NBLK))


_mlp_call = pl.pallas_call(
    _mlp_body,
    grid=(_NBLK,),
    in_specs=[
        pl.BlockSpec((_GR, 4 * HIDDEN_DIM), lambda i: (i, 0)),
        pl.BlockSpec((_GR, 4 * HIDDEN_DIM), lambda i: (i, 0)),
        _espec(0), _espec(1), _espec(2), _espec(3),
        pl.BlockSpec((4 * HIDDEN_DIM, 4 * HIDDEN_DIM), lambda i: (0, 0)),
        pl.BlockSpec((HIDDEN_DIM, EDGE_DIM), lambda i: (0, 0)),
        pl.BlockSpec((HIDDEN_DIM, 1), lambda i: (0, 0)),
        pl.BlockSpec((EDGE_DIM, HIDDEN_DIM), lambda i: (0, 0)),
        pl.BlockSpec((EDGE_DIM, 1), lambda i: (0, 0)),
    ],
    out_specs=[pl.BlockSpec((EDGE_DIM, _GR), lambda i: (0, i))] * 4,
    out_shape=[jax.ShapeDtypeStruct((EDGE_DIM, QROWS), jnp.float32)] * 4,
)


def kernel(node_features, edge_features, edge_index, W1, b1, W2, b2):
    src = edge_index[0].astype(jnp.int32)
    dst = edge_index[1].astype(jnp.int32)
    pa, pb = _proj_call(node_features, W1[:NODE_DIM], W1[NODE_DIM:2 * NODE_DIM])
    ga, gb = _gather2(pa, pb, src, dst)

    et = edge_features.T
    eye = jnp.eye(4 * HIDDEN_DIM, dtype=jnp.float32)
    w1et = W1[2 * NODE_DIM:].T
    w2t = W2.T
    outs = _mlp_call(ga, gb, et, et, et, et,
                     eye, w1et, b1.reshape(HIDDEN_DIM, 1),
                     w2t, b2.reshape(EDGE_DIM, 1))
    out_t = jnp.concatenate(outs, axis=1)
    return out_t.T


# speedup vs baseline: 1.1078x; 1.1078x over previous
import functools

import jax
import jax.numpy as jnp
from jax import lax
from jax.experimental import pallas as pl
from jax.experimental.pallas import tpu as pltpu
from jax.experimental.pallas import tpu_sc as plsc

N_NODES = 10000
N_EDGES = 320000
NODE_DIM = 128
EDGE_DIM = 16
HIDDEN_DIM = 32

NC = 2
NS = 16
NW = NC * NS
CH = 128
NT = N_EDGES // CH
MAXCH = 79
IDX_PRE = MAXCH * CH
NBUF = 8
AHEAD = 4
NOUTER = 10
QROWS = N_EDGES // 4
QTILES = NT // 4


def _proj_body(nf_ref, wa_ref, wb_ref, pa_ref, pb_ref):
    nf = nf_ref[...]
    pa_ref[...] = jnp.dot(nf, wa_ref[...], preferred_element_type=jnp.float32)
    pb_ref[...] = jnp.dot(nf, wb_ref[...], preferred_element_type=jnp.float32)


_proj_call = pl.pallas_call(
    _proj_body,
    out_shape=[
        jax.ShapeDtypeStruct((N_NODES, HIDDEN_DIM), jnp.float32),
        jax.ShapeDtypeStruct((N_NODES, HIDDEN_DIM), jnp.float32),
    ],
)


_mesh = plsc.VectorSubcoreMesh(
    core_axis_name="c", subcore_axis_name="s", num_cores=NC, num_subcores=NS
)


@functools.partial(
    pl.kernel,
    out_type=jax.ShapeDtypeStruct((QROWS, 4 * HIDDEN_DIM), jnp.float32),
    mesh=_mesh,
    scratch_types=[
        pltpu.VMEM((IDX_PRE,), jnp.int32),
        pltpu.VMEM((IDX_PRE,), jnp.int32),
        pltpu.VMEM((NBUF, CH, HIDDEN_DIM), jnp.float32),
        [pltpu.SemaphoreType.DMA] * NBUF,
        [pltpu.SemaphoreType.DMA] * NBUF,
        [pltpu.SemaphoreType.DMA] * NBUF,
    ],
    compiler_params=pltpu.CompilerParams(use_tc_tiling_on_sc=False,
                                         needs_layout_passes=False),
)
def _gather2(pa_hbm, pb_hbm, src_hbm, dst_hbm, g_hbm,
             src_all, dst_all, a_v, gasems, gbsems, ssems):
    wid = lax.axis_index("s") * NC + lax.axis_index("c")
    t0 = wid * NT // NW
    t1 = (wid + 1) * NT // NW
    nch = t1 - t0
    q = wid // 8
    r0base = (t0 - q * QTILES) * CH

    pltpu.sync_copy(src_hbm.at[pl.ds(t0 * CH, IDX_PRE)], src_all)
    pltpu.sync_copy(dst_hbm.at[pl.ds(t0 * CH, IDX_PRE)], dst_all)

    def ga_issue(i, b):
        pltpu.async_copy(
            pa_hbm.at[src_all.at[pl.ds(i * CH, CH)]], a_v.at[b], gasems[b])

    def ga_wait(i, b):
        pltpu.make_async_copy(
            pa_hbm.at[src_all.at[pl.ds(i * CH, CH)]], a_v.at[b],
            gasems[b]).wait()

    def gb_issue(i, b):
        pltpu.async_copy(
            pb_hbm.at[dst_all.at[pl.ds(i * CH, CH)]], a_v.at[b], gbsems[b],
            add=True)

    def gb_wait(i, b):
        pltpu.make_async_copy(
            pb_hbm.at[dst_all.at[pl.ds(i * CH, CH)]], a_v.at[b],
            gbsems[b]).wait()

    def store(i, b):
        r0 = r0base + i * CH
        col = pl.ds(q * HIDDEN_DIM, HIDDEN_DIM)
        return pltpu.make_async_copy(
            a_v.at[b], g_hbm.at[pl.ds(r0, CH), col], ssems[b])

    for b in range(AHEAD):
        ga_issue(b, b)
    for b in range(2):
        ga_wait(b, b)
        gb_issue(b, b)

    def outer(t, carry):
        for b in range(NBUF):
            i = t * NBUF + b

            @pl.when(i < nch)
            def _():
                gb_wait(i, b)
                store(i, b).start()

            @pl.when(i + 2 < nch)
            def _():
                b2 = (b + 2) % NBUF
                ga_wait(i + 2, b2)
                gb_issue(i + 2, b2)

            @pl.when(i + AHEAD < nch)
            def _():
                b4 = (b + AHEAD) % NBUF

                @pl.when(i >= AHEAD)
                def _():
                    store(i - AHEAD, b4).wait()

                ga_issue(i + AHEAD, b4)

        return carry

    lax.fori_loop(0, NOUTER, outer, 0)

    for b in range(NBUF):
        store(0, b).wait()


_GR = 3200
_NBLK = QROWS // _GR


def _mlp_body(g_ref, e0_ref, e1_ref, e2_ref, e3_ref, eye_ref,
              w1et_ref, b1_ref, w2t_ref, b2_ref,
              o0_ref, o1_ref, o2_ref, o3_ref):
    gt = g_ref[...].T
    w1et = w1et_ref[...]
    w2t = w2t_ref[...]
    b1 = b1_ref[...]
    b2 = b2_ref[...]
    for c, (e_ref, o_ref) in enumerate(
            [(e0_ref, o0_ref), (e1_ref, o1_ref), (e2_ref, o2_ref),
             (e3_ref, o3_ref)]):
        ec = e_ref[...]
        pre = gt[c * HIDDEN_DIM:(c + 1) * HIDDEN_DIM, :] + jnp.dot(
            w1et, ec, preferred_element_type=jnp.float32) + b1
        h = jnp.maximum(pre, 0.0)
        o_ref[...] = ec + jnp.dot(w2t, h,
                                  preferred_element_type=jnp.float32) + b2


def _espec(c):
    return pl.BlockSpec((EDGE_DIM, _GR), lambda i, c=c: (0, i + c * _NBLK))


_mlp_call = pl.pallas_call(
    _mlp_body,
    grid=(_NBLK,),
    in_specs=[
        pl.BlockSpec((_GR, 4 * HIDDEN_DIM), lambda i: (i, 0)),
        _espec(0), _espec(1), _espec(2), _espec(3),
        pl.BlockSpec((4 * HIDDEN_DIM, 4 * HIDDEN_DIM), lambda i: (0, 0)),
        pl.BlockSpec((HIDDEN_DIM, EDGE_DIM), lambda i: (0, 0)),
        pl.BlockSpec((HIDDEN_DIM, 1), lambda i: (0, 0)),
        pl.BlockSpec((EDGE_DIM, HIDDEN_DIM), lambda i: (0, 0)),
        pl.BlockSpec((EDGE_DIM, 1), lambda i: (0, 0)),
    ],
    out_specs=[pl.BlockSpec((EDGE_DIM, _GR), lambda i: (0, i))] * 4,
    out_shape=[jax.ShapeDtypeStruct((EDGE_DIM, QROWS), jnp.float32)] * 4,
)


def kernel(node_features, edge_features, edge_index, W1, b1, W2, b2):
    src = edge_index[0].astype(jnp.int32)
    dst = edge_index[1].astype(jnp.int32)
    pa, pb = _proj_call(node_features, W1[:NODE_DIM], W1[NODE_DIM:2 * NODE_DIM])
    g = _gather2(pa, pb, src, dst)

    et = edge_features.T
    eye = jnp.eye(4 * HIDDEN_DIM, dtype=jnp.float32)
    w1et = W1[2 * NODE_DIM:].T
    w2t = W2.T
    outs = _mlp_call(g, et, et, et, et,
                     eye, w1et, b1.reshape(HIDDEN_DIM, 1),
                     w2t, b2.reshape(EDGE_DIM, 1))
    out_t = jnp.concatenate(outs, axis=1)
    return out_t.T
